# baseline (speedup 1.0000x reference)
import functools

import jax
import jax.numpy as jnp
from jax import lax
from jax.experimental import pallas as pl
from jax.experimental.pallas import tpu as pltpu
from jax.experimental.pallas import tpu_sc as plsc

NUM_FIELDS = 26
VOCAB = 100000
EMB = 32
BATCH = 16384

NC = 2
NS = 16
NW = NC * NS
ROWS_PER_W = BATCH // NW
IDX_PER_W = NUM_FIELDS * ROWS_PER_W
LANES = 16
GROUP = 8
CHUNK = 16
N_ITEMS = IDX_PER_W // CHUNK
NCH = ROWS_PER_W // CHUNK
ITEM_BYTES = CHUNK * GROUP * EMB * 4


def _make_kernel():
    mesh = plsc.VectorSubcoreMesh(core_axis_name="c", subcore_axis_name="s")

    @functools.partial(
        pl.kernel,
        out_type=jax.ShapeDtypeStruct((BATCH * EMB,), jnp.float32),
        mesh=mesh,
        compiler_params=pltpu.CompilerParams(use_tc_tiling_on_sc=True),
        scratch_types=[
            pltpu.VMEM((IDX_PER_W,), jnp.int32),
            pltpu.VMEM((2, CHUNK, GROUP, EMB), jnp.float32),
            pltpu.VMEM((ROWS_PER_W * EMB,), jnp.float32),
            pltpu.VMEM((CHUNK * GROUP, EMB), jnp.float32),
            pltpu.SemaphoreType.DMA((2,)),
        ],
    )
    def k(tab_hbm, xf_hbm, out_hbm, idx, buf, acc, dummy, sem):
        wid = lax.axis_index("s") * NC + lax.axis_index("c")

        pltpu.sync_copy(xf_hbm.at[pl.ds(wid * IDX_PER_W, IDX_PER_W)], idx)

        zeros = jnp.zeros((LANES,), jnp.float32)

        @pl.loop(0, ROWS_PER_W * EMB // LANES)
        def _zero(c):
            acc[pl.ds(c * LANES, LANES)] = zeros

        def issue(i, slot):
            f = i // NCH
            vraw = idx[pl.ds(i * CHUNK, CHUNK)]
            for j in range(CHUNK):
                g8 = pl.multiple_of(vraw[j] & -GROUP, GROUP)
                pltpu.async_copy(
                    tab_hbm.at[f, pl.ds(g8, GROUP), :],
                    buf.at[slot, j],
                    sem.at[slot],
                )

        issue(0, 0)

        @pl.loop(0, N_ITEMS)
        def _item(i):
            slot = i & 1

            @pl.when(i + 1 < N_ITEMS)
            def _():
                issue(i + 1, 1 - slot)

            pltpu.make_async_copy(
                tab_hbm.at[0, pl.ds(0, CHUNK * GROUP), :],
                dummy,
                sem.at[slot],
            ).wait()

            g = i & (NCH - 1)
            vraw = idx[pl.ds(i * CHUNK, CHUNK)]
            for j in range(CHUNK):
                w = vraw[j] & (GROUP - 1)
                for half in range(EMB // LANES):
                    plsc.addupdate(
                        acc.at[pl.ds((g * CHUNK + j) * EMB + half * LANES, LANES)],
                        buf[slot, j, w, pl.ds(half * LANES, LANES)],
                    )

        pltpu.sync_copy(acc, out_hbm.at[pl.ds(wid * ROWS_PER_W * EMB, ROWS_PER_W * EMB)])

    return k


_sc_kernel = _make_kernel()


@jax.jit
def kernel(x, tables):
    xf = (
        x.astype(jnp.int32)
        .reshape(NW, ROWS_PER_W, NUM_FIELDS)
        .transpose(0, 2, 1)
        .reshape(-1)
    )
    out = _sc_kernel(tables, xf)
    return out.reshape(BATCH, EMB)
